# Initial kernel scaffold; baseline (speedup 1.0000x reference)
import jax
import jax.numpy as jnp
from jax.experimental import pallas as pl


def kernel(x, edge_index, edge_attr, W, b):
    raise NotImplementedError("write your pallas kernel here")


# speedup vs baseline: 103.7575x; 103.7575x over previous
import functools

import jax
import jax.numpy as jnp
from jax import lax
from jax.experimental import pallas as pl
from jax.experimental.pallas import tpu as pltpu
from jax.experimental.pallas import tpu_sc as plsc

N_NODES = 10000
D_FEAT = 128
N_EDGES = 320000

NPAD = 10240
NROWS = NPAD // 128
EPAD = 327680
EC = EPAD // 32
RC = EC // 128
NSLC = NPAD // 16

_mesh = plsc.VectorSubcoreMesh(core_axis_name="c", subcore_axis_name="s")


def _zero_shared(zbuf, shared, s):
    def zb(i, carry):
        zbuf[pl.ds(i * 16, 16)] = jnp.zeros((16,), jnp.float32)
        return carry

    lax.fori_loop(0, NSLC // 16, zb, 0)
    pltpu.sync_copy(zbuf, shared.at[pl.ds(s * NSLC, NSLC)])


@functools.partial(
    pl.kernel,
    out_type=jax.ShapeDtypeStruct((2, 16, NSLC), jnp.float32),
    mesh=_mesh,
    scratch_types=[
        pltpu.VMEM((RC, 128), jnp.int32),
        pltpu.VMEM((EC,), jnp.float32),
        pltpu.VMEM_SHARED((NPAD,), jnp.float32),
        pltpu.VMEM((NSLC,), jnp.float32),
        pltpu.SemaphoreType.DMA,
    ],
)
def _sc_deg(col2_hbm, ew_hbm, degp_hbm, colv, ewv, shared, zbuf, sem):
    c = lax.axis_index("c")
    s = lax.axis_index("s")
    t = c * 16 + s
    _zero_shared(zbuf, shared, s)
    pltpu.sync_copy(col2_hbm.at[pl.ds(t * RC, RC)], colv)
    pltpu.sync_copy(ew_hbm.at[pl.ds(t * EC, EC)], ewv)
    plsc.subcore_barrier()
    cps = [
        pltpu.async_copy(
            ewv.at[pl.ds(j * 128, 128)], shared.at[colv.at[j]], sem, add=True
        )
        for j in range(RC)
    ]
    for cp in cps:
        cp.wait()
    plsc.subcore_barrier()
    pltpu.sync_copy(shared.at[pl.ds(s * NSLC, NSLC)], degp_hbm.at[c, s])


@functools.partial(
    pl.kernel,
    out_type=jax.ShapeDtypeStruct((2, 16, NSLC), jnp.float32),
    mesh=_mesh,
    scratch_types=[
        pltpu.VMEM((EC,), jnp.int32),
        pltpu.VMEM((RC, 128), jnp.int32),
        pltpu.VMEM((EC,), jnp.float32),
        pltpu.VMEM((NPAD,), jnp.float32),
        pltpu.VMEM_SHARED((NPAD,), jnp.float32),
        pltpu.VMEM((NSLC,), jnp.float32),
        pltpu.SemaphoreType.DMA,
    ],
    compiler_params=pltpu.CompilerParams(needs_layout_passes=False),
)
def _sc_msg(row_hbm, col2_hbm, ew_hbm, g_hbm, sp_hbm,
            rowv, colv, ewv, gv, shared, zbuf, sem):
    c = lax.axis_index("c")
    s = lax.axis_index("s")
    t = c * 16 + s
    _zero_shared(zbuf, shared, s)
    pltpu.sync_copy(g_hbm, gv)
    pltpu.sync_copy(row_hbm.at[pl.ds(t * EC, EC)], rowv)
    pltpu.sync_copy(col2_hbm.at[pl.ds(t * RC, RC)], colv)
    pltpu.sync_copy(ew_hbm.at[pl.ds(t * EC, EC)], ewv)
    plsc.subcore_barrier()

    def body(i, carry):
        sl = pl.ds(i * 16, 16)
        ii = rowv[sl]
        vals = plsc.load_gather(gv, [ii])
        ewv[sl] = vals * ewv[sl]
        return carry

    lax.fori_loop(0, EC // 16, body, 0)
    cps = [
        pltpu.async_copy(
            ewv.at[pl.ds(j * 128, 128)], shared.at[colv.at[j]], sem, add=True
        )
        for j in range(RC)
    ]
    for cp in cps:
        cp.wait()
    plsc.subcore_barrier()
    pltpu.sync_copy(shared.at[pl.ds(s * NSLC, NSLC)], sp_hbm.at[c, s])


def _tc_prep_body(x_ref, w_ref, degp_ref, b_ref, g_ref, dinv_ref, self_ref):
    x = x_ref[...]
    w = w_ref[...]
    h = jnp.sum(x * w[0], axis=2)
    deg = 1.0 + degp_ref[0] + degp_ref[1]
    dinv = lax.rsqrt(deg)
    g_ref[...] = h * dinv
    dinv_ref[...] = dinv
    self_ref[...] = h / deg + b_ref[...]


def _tc_final_body(sp_ref, dinv_ref, self_ref, out_ref):
    v = (sp_ref[0] + sp_ref[1]) * dinv_ref[...] + self_ref[...]
    sp = jnp.maximum(v, 0.0) + jnp.log1p(jnp.exp(-jnp.abs(v)))
    out_ref[...] = v * jnp.tanh(sp)


def kernel(x, edge_index, edge_attr, W, b):
    ei = edge_index.astype(jnp.int32)
    row = ei[0]
    col = ei[1]
    ea = edge_attr.astype(jnp.float32)

    pad = EPAD - N_EDGES
    padidx = jnp.arange(pad, dtype=jnp.int32)
    row_p = jnp.concatenate([row, padidx])
    col2 = jnp.concatenate([col, padidx]).reshape(EPAD // 128, 128)
    ea_p = jnp.concatenate([ea, jnp.zeros((pad,), jnp.float32)])

    degp = _sc_deg(col2, ea_p)

    x3 = jnp.concatenate(
        [x, jnp.zeros((NPAD - N_NODES, D_FEAT), jnp.float32)]
    ).reshape(NROWS, 128, 128)
    w2 = W.reshape(1, 128)
    b2 = jnp.broadcast_to(b.reshape(1, 1), (1, 128))

    g, dinv, selfterm = pl.pallas_call(
        _tc_prep_body,
        out_shape=[jax.ShapeDtypeStruct((NROWS, 128), jnp.float32)] * 3,
    )(x3, w2, degp.reshape(2, NROWS, 128), b2)

    sp = _sc_msg(row_p, col2, ea_p, g.reshape(NPAD))

    out = pl.pallas_call(
        _tc_final_body,
        out_shape=jax.ShapeDtypeStruct((NROWS, 128), jnp.float32),
    )(sp.reshape(2, NROWS, 128), dinv, selfterm)

    return out.reshape(1, NPAD)[:, :N_NODES]


# speedup vs baseline: 108.8184x; 1.0488x over previous
import functools

import jax
import jax.numpy as jnp
from jax import lax
from jax.experimental import pallas as pl
from jax.experimental.pallas import tpu as pltpu
from jax.experimental.pallas import tpu_sc as plsc

N_NODES = 10000
D_FEAT = 128
N_EDGES = 320000

NPAD = 10240
NROWS = NPAD // 128
EC = N_EDGES // 32
ECP = 10112
RC = ECP // 128
NV = EC // 16
NSLC = NPAD // 16

_mesh = plsc.VectorSubcoreMesh(core_axis_name="c", subcore_axis_name="s")


def _zero_shared(zbuf, shared, s):
    @plsc.parallel_loop(0, NSLC // 16, unroll=4)
    def _zb(i):
        zbuf[pl.ds(i * 16, 16)] = jnp.zeros((16,), jnp.float32)

    pltpu.sync_copy(zbuf, shared.at[pl.ds(s * NSLC, NSLC)])


def _pad_tail(ewv, col2d, t):
    lanes = jnp.arange(16, dtype=jnp.int32)
    for m in range(7):
        ewv[pl.ds(EC + m * 16, 16)] = jnp.zeros((16,), jnp.float32)
        col2d[RC - 1, pl.ds((m + 1) * 16, 16)] = t * 112 + m * 16 + lanes


def _scatter_chunks(ewv, col2d, shared, sem):
    cps = [
        pltpu.async_copy(
            ewv.at[pl.ds(j * 128, 128)], shared.at[col2d.at[j]], sem, add=True
        )
        for j in range(RC)
    ]
    for cp in cps:
        cp.wait()


@functools.partial(
    pl.kernel,
    out_type=jax.ShapeDtypeStruct((2, 16, NSLC), jnp.float32),
    mesh=_mesh,
    scratch_types=[
        pltpu.VMEM((EC,), jnp.int32),
        pltpu.VMEM((RC, 128), jnp.int32),
        pltpu.VMEM((ECP,), jnp.float32),
        pltpu.VMEM_SHARED((NPAD,), jnp.float32),
        pltpu.VMEM((NSLC,), jnp.float32),
        pltpu.SemaphoreType.DMA,
    ],
    compiler_params=pltpu.CompilerParams(needs_layout_passes=False),
)
def _sc_deg(col_hbm, ew_hbm, degp_hbm, colv, col2d, ewv, shared, zbuf, sem):
    c = lax.axis_index("c")
    s = lax.axis_index("s")
    t = c * 16 + s
    _zero_shared(zbuf, shared, s)
    pltpu.sync_copy(col_hbm.at[pl.ds(t * EC, EC)], colv)
    pltpu.sync_copy(ew_hbm.at[pl.ds(t * EC, EC)], ewv.at[pl.ds(0, EC)])
    _pad_tail(ewv, col2d, t)

    @plsc.parallel_loop(0, NV, unroll=8)
    def _rw(i):
        col2d[i >> 3, pl.ds((i & 7) * 16, 16)] = colv[pl.ds(i * 16, 16)]

    plsc.subcore_barrier()
    _scatter_chunks(ewv, col2d, shared, sem)
    plsc.subcore_barrier()
    pltpu.sync_copy(shared.at[pl.ds(s * NSLC, NSLC)], degp_hbm.at[c, s])


@functools.partial(
    pl.kernel,
    out_type=jax.ShapeDtypeStruct((2, 16, NSLC), jnp.float32),
    mesh=_mesh,
    scratch_types=[
        pltpu.VMEM((EC,), jnp.int32),
        pltpu.VMEM((EC,), jnp.int32),
        pltpu.VMEM((RC, 128), jnp.int32),
        pltpu.VMEM((ECP,), jnp.float32),
        pltpu.VMEM((NPAD,), jnp.float32),
        pltpu.VMEM_SHARED((NPAD,), jnp.float32),
        pltpu.VMEM((NSLC,), jnp.float32),
        pltpu.SemaphoreType.DMA,
    ],
    compiler_params=pltpu.CompilerParams(needs_layout_passes=False),
)
def _sc_msg(row_hbm, col_hbm, ew_hbm, g_hbm, sp_hbm,
            rowv, colv, col2d, ewv, gv, shared, zbuf, sem):
    c = lax.axis_index("c")
    s = lax.axis_index("s")
    t = c * 16 + s
    _zero_shared(zbuf, shared, s)
    pltpu.sync_copy(g_hbm, gv)
    pltpu.sync_copy(row_hbm.at[pl.ds(t * EC, EC)], rowv)
    pltpu.sync_copy(col_hbm.at[pl.ds(t * EC, EC)], colv)
    pltpu.sync_copy(ew_hbm.at[pl.ds(t * EC, EC)], ewv.at[pl.ds(0, EC)])
    _pad_tail(ewv, col2d, t)

    @plsc.parallel_loop(0, NV, unroll=8)
    def _gm(i):
        sl = pl.ds(i * 16, 16)
        ii = rowv[sl]
        vals = plsc.load_gather(gv, [ii])
        ewv[sl] = vals * ewv[sl]
        col2d[i >> 3, pl.ds((i & 7) * 16, 16)] = colv[sl]

    plsc.subcore_barrier()
    _scatter_chunks(ewv, col2d, shared, sem)
    plsc.subcore_barrier()
    pltpu.sync_copy(shared.at[pl.ds(s * NSLC, NSLC)], sp_hbm.at[c, s])


def _tc_prep_body(x_ref, w_ref, degp_ref, b_ref, g_ref, dinv_ref, self_ref):
    x = x_ref[...]
    w = w_ref[...]
    h = jnp.sum(x * w[0], axis=2)
    deg = 1.0 + degp_ref[0] + degp_ref[1]
    dinv = lax.rsqrt(deg)
    g_ref[...] = h * dinv
    dinv_ref[...] = dinv
    self_ref[...] = h / deg + b_ref[...]


def _tc_final_body(sp_ref, dinv_ref, self_ref, out_ref):
    v = (sp_ref[0] + sp_ref[1]) * dinv_ref[...] + self_ref[...]
    sp = jnp.maximum(v, 0.0) + jnp.log1p(jnp.exp(-jnp.abs(v)))
    out_ref[...] = v * jnp.tanh(sp)


def kernel(x, edge_index, edge_attr, W, b):
    ei = edge_index.astype(jnp.int32)
    row = ei[0]
    col = ei[1]
    ea = edge_attr.astype(jnp.float32)

    degp = _sc_deg(col, ea)

    x3 = jnp.concatenate(
        [x, jnp.zeros((NPAD - N_NODES, D_FEAT), jnp.float32)]
    ).reshape(NROWS, 128, 128)
    w2 = W.reshape(1, 128)
    b2 = jnp.broadcast_to(b.reshape(1, 1), (1, 128))

    g, dinv, selfterm = pl.pallas_call(
        _tc_prep_body,
        out_shape=[jax.ShapeDtypeStruct((NROWS, 128), jnp.float32)] * 3,
    )(x3, w2, degp.reshape(2, NROWS, 128), b2)

    sp = _sc_msg(row, col, ea, g.reshape(NPAD))

    out = pl.pallas_call(
        _tc_final_body,
        out_shape=jax.ShapeDtypeStruct((NROWS, 128), jnp.float32),
    )(sp.reshape(2, NROWS, 128), dinv, selfterm)

    return out.reshape(1, NPAD)[:, :N_NODES]


# speedup vs baseline: 154.5061x; 1.4199x over previous
import functools

import jax
import jax.numpy as jnp
from jax import lax
from jax.experimental import pallas as pl
from jax.experimental.pallas import tpu as pltpu
from jax.experimental.pallas import tpu_sc as plsc

N_NODES = 10000
D_FEAT = 128
N_EDGES = 320000

NPAD = 10240
NROWS = NPAD // 128
EC = N_EDGES // 32
ECP = 10112
RC = ECP // 128
NV = EC // 16
NSLC = NPAD // 16

_mesh = plsc.VectorSubcoreMesh(core_axis_name="c", subcore_axis_name="s")


def _zero_shared(zbuf, shared, s):
    @plsc.parallel_loop(0, NSLC // 16, unroll=4)
    def _zb(i):
        zbuf[pl.ds(i * 16, 16)] = jnp.zeros((16,), jnp.float32)

    pltpu.sync_copy(zbuf, shared.at[pl.ds(s * NSLC, NSLC)])


def _pad_tail(ewv, col2d, t):
    lanes = jnp.arange(16, dtype=jnp.int32)
    for m in range(7):
        ewv[pl.ds(EC + m * 16, 16)] = jnp.zeros((16,), jnp.float32)
        col2d[RC - 1, pl.ds((m + 1) * 16, 16)] = t * 112 + m * 16 + lanes


def _scatter_chunks(ewv, col2d, shared, sem):
    cps = [
        pltpu.async_copy(
            ewv.at[pl.ds(j * 128, 128)], shared.at[col2d.at[j]], sem, add=True
        )
        for j in range(RC)
    ]
    for cp in cps:
        cp.wait()


def _writeout(shared, out0, out1, c, s):
    @pl.when(c == 0)
    def _():
        pltpu.sync_copy(shared.at[pl.ds(s * NSLC, NSLC)],
                        out0.at[pl.ds(s * NSLC, NSLC)])

    @pl.when(c == 1)
    def _():
        pltpu.sync_copy(shared.at[pl.ds(s * NSLC, NSLC)],
                        out1.at[pl.ds(s * NSLC, NSLC)])


@functools.partial(
    pl.kernel,
    out_type=[jax.ShapeDtypeStruct((NPAD,), jnp.float32)] * 2,
    mesh=_mesh,
    scratch_types=[
        pltpu.VMEM((EC,), jnp.int32),
        pltpu.VMEM((RC, 128), jnp.int32),
        pltpu.VMEM((ECP,), jnp.float32),
        pltpu.VMEM_SHARED((NPAD,), jnp.float32),
        pltpu.VMEM((NSLC,), jnp.float32),
        pltpu.SemaphoreType.DMA,
    ],
    compiler_params=pltpu.CompilerParams(needs_layout_passes=False),
)
def _sc_deg(ei_hbm, ew_hbm, degp0, degp1, colv, col2d, ewv, shared, zbuf, sem):
    c = lax.axis_index("c")
    s = lax.axis_index("s")
    t = c * 16 + s
    _zero_shared(zbuf, shared, s)
    pltpu.sync_copy(ei_hbm.at[pl.ds(N_EDGES + t * EC, EC)], colv)
    pltpu.sync_copy(ew_hbm.at[pl.ds(t * EC, EC)], ewv.at[pl.ds(0, EC)])
    _pad_tail(ewv, col2d, t)

    @plsc.parallel_loop(0, NV, unroll=8)
    def _rw(i):
        col2d[i >> 3, pl.ds((i & 7) * 16, 16)] = colv[pl.ds(i * 16, 16)]

    plsc.subcore_barrier()
    _scatter_chunks(ewv, col2d, shared, sem)
    plsc.subcore_barrier()
    _writeout(shared, degp0, degp1, c, s)


@functools.partial(
    pl.kernel,
    out_type=[jax.ShapeDtypeStruct((NPAD,), jnp.float32)] * 2,
    mesh=_mesh,
    scratch_types=[
        pltpu.VMEM((EC,), jnp.int32),
        pltpu.VMEM((EC,), jnp.int32),
        pltpu.VMEM((RC, 128), jnp.int32),
        pltpu.VMEM((ECP,), jnp.float32),
        pltpu.VMEM((NPAD,), jnp.float32),
        pltpu.VMEM_SHARED((NPAD,), jnp.float32),
        pltpu.VMEM((NSLC,), jnp.float32),
        pltpu.SemaphoreType.DMA,
    ],
    compiler_params=pltpu.CompilerParams(needs_layout_passes=False),
)
def _sc_msg(ei_hbm, ew_hbm, g_hbm, sp0, sp1,
            rowv, colv, col2d, ewv, gv, shared, zbuf, sem):
    c = lax.axis_index("c")
    s = lax.axis_index("s")
    t = c * 16 + s
    _zero_shared(zbuf, shared, s)
    pltpu.sync_copy(g_hbm, gv)
    pltpu.sync_copy(ei_hbm.at[pl.ds(t * EC, EC)], rowv)
    pltpu.sync_copy(ei_hbm.at[pl.ds(N_EDGES + t * EC, EC)], colv)
    pltpu.sync_copy(ew_hbm.at[pl.ds(t * EC, EC)], ewv.at[pl.ds(0, EC)])
    _pad_tail(ewv, col2d, t)

    @plsc.parallel_loop(0, NV, unroll=8)
    def _gm(i):
        sl = pl.ds(i * 16, 16)
        ii = rowv[sl]
        vals = plsc.load_gather(gv, [ii])
        ewv[sl] = vals * ewv[sl]
        col2d[i >> 3, pl.ds((i & 7) * 16, 16)] = colv[sl]

    plsc.subcore_barrier()
    _scatter_chunks(ewv, col2d, shared, sem)
    plsc.subcore_barrier()
    _writeout(shared, sp0, sp1, c, s)


def _tc_mm_body(x_ref, w_ref, h_ref):
    h_ref[...] = lax.dot_general(x_ref[...], w_ref[...],
                                 (((2,), (0,)), ((), ())),
                                 preferred_element_type=jnp.float32)


def _tc_prep_body(h_ref, d0_ref, d1_ref, b_ref, g_ref, dinv_ref, self_ref):
    h = h_ref[...]
    deg = 1.0 + d0_ref[...] + d1_ref[...]
    dinv = lax.rsqrt(deg)
    g_ref[...] = h * dinv
    dinv_ref[...] = dinv
    self_ref[...] = h / deg + b_ref[...]


def _tc_final_body(s0_ref, s1_ref, dinv_ref, self_ref, out_ref):
    v = (s0_ref[...] + s1_ref[...]) * dinv_ref[...] + self_ref[...]
    sp = jnp.maximum(v, 0.0) + jnp.log1p(jnp.exp(-jnp.abs(v)))
    out_ref[...] = v * jnp.tanh(sp)


def kernel(x, edge_index, edge_attr, W, b):
    eflat = edge_index.astype(jnp.int32).reshape(2 * N_EDGES)
    ea = edge_attr.astype(jnp.float32)

    degp0, degp1 = _sc_deg(eflat, ea)

    x3 = jnp.concatenate(
        [x, jnp.zeros((NPAD - N_NODES, D_FEAT), jnp.float32)]
    ).reshape(NROWS, 128, 128)
    w1 = W.reshape(128)
    b2 = jnp.broadcast_to(b.reshape(1, 1), (1, 128))

    h3 = pl.pallas_call(
        _tc_mm_body,
        out_shape=jax.ShapeDtypeStruct((NROWS, 128), jnp.float32),
    )(x3, w1)

    g, dinv, selfterm = pl.pallas_call(
        _tc_prep_body,
        out_shape=[jax.ShapeDtypeStruct((NROWS, 128), jnp.float32)] * 3,
    )(h3, degp0.reshape(NROWS, 128), degp1.reshape(NROWS, 128), b2)

    sp0, sp1 = _sc_msg(eflat, ea, g.reshape(NPAD))

    out = pl.pallas_call(
        _tc_final_body,
        out_shape=jax.ShapeDtypeStruct((NROWS, 128), jnp.float32),
    )(sp0.reshape(NROWS, 128), sp1.reshape(NROWS, 128), dinv, selfterm)

    return out.reshape(1, NPAD)[:, :N_NODES]
